# baseline (speedup 1.0000x reference)
import functools

import jax
import jax.numpy as jnp
from jax import lax
from jax.experimental import pallas as pl
from jax.experimental.pallas import tpu as pltpu
from jax.experimental.pallas import tpu_sc as plsc

F = 128
BN = 512
K = 128
Q = 5


def _sc_edge_pass(h, src2, dst2, dummy, n_pad, with_deg):
  info = plsc.get_sparse_core_info()
  nc, ns = info.num_cores, info.num_subcores
  nw = nc * ns
  e_pad = src2.shape[0] * K
  epw = e_pad // nw
  rpw = epw // K
  rpq = rpw // Q
  pairs = rpq // 2
  rpt = n_pad // ns
  mesh = plsc.VectorSubcoreMesh(core_axis_name="c", subcore_axis_name="s")

  out_type = [jax.ShapeDtypeStruct((nc * n_pad, F), jnp.float32)]
  if with_deg:
    out_type.append(jax.ShapeDtypeStruct((nc * n_pad,), jnp.float32))

  scratch = [
      pltpu.VMEM((rpq, K), jnp.int32),
      pltpu.VMEM((rpq, K), jnp.int32),
      pltpu.VMEM((K, F), jnp.float32),
      pltpu.VMEM((K, F), jnp.float32),
      pltpu.VMEM((K,), jnp.float32),
      pltpu.VMEM((rpt,), jnp.float32),
      pltpu.VMEM_SHARED((n_pad, F), jnp.float32),
      pltpu.VMEM_SHARED((n_pad,), jnp.float32),
      pltpu.SemaphoreType.DMA,
      pltpu.SemaphoreType.DMA,
      pltpu.SemaphoreType.DMA,
      pltpu.SemaphoreType.DMA,
  ]

  def body(h_hbm, src_hbm, dst_hbm, dummy_hbm, agg_out, *rest):
    if with_deg:
      deg_out = rest[0]
      rest = rest[1:]
    (idx_sa, idx_da, rows0, rows1, ones, zdeg, agg_sh, deg_sh,
     gsem0, gsem1, ssem0, ssem1) = rest
    rows = (rows0, rows1)
    gsem = (gsem0, gsem1)
    ssem = (ssem0, ssem1)
    cid = lax.axis_index("c")
    sid = lax.axis_index("s")
    wid = sid * nc + cid

    zv = jnp.zeros((16,), jnp.float32)

    def z16(i, carry):
      rows0[i // (F // 16), pl.ds((i % (F // 16)) * 16, 16)] = zv
      return carry

    lax.fori_loop(0, K * (F // 16), z16, 0)
    for t in range(rpt // K):
      pltpu.sync_copy(rows0, agg_sh.at[pl.ds(sid * rpt + t * K, K)])

    if with_deg:
      ov = jnp.ones((16,), jnp.float32)

      def o16(i, carry):
        ones[pl.ds(i * 16, 16)] = ov
        return carry

      lax.fori_loop(0, K // 16, o16, 0)

      def zd16(i, carry):
        zdeg[pl.ds(i * 16, 16)] = zv
        return carry

      lax.fori_loop(0, rpt // 16, zd16, 0)
      pltpu.sync_copy(zdeg, deg_sh.at[pl.ds(sid * rpt, rpt)])

    plsc.subcore_barrier()

    def do_block(b, carry):
      pltpu.sync_copy(h_hbm.at[idx_sa.at[b]], rows0)
      pltpu.sync_copy(rows0, agg_sh.at[idx_da.at[b]], add=True)
      if with_deg:
        pltpu.sync_copy(ones, deg_sh.at[idx_da.at[b]], add=True)
      return carry

    for q in range(Q):
      qrow = wid * rpw + q * rpq
      pltpu.sync_copy(src_hbm.at[pl.ds(qrow, rpq)], idx_sa)
      pltpu.sync_copy(dst_hbm.at[pl.ds(qrow, rpq)], idx_da)
      lax.fori_loop(0, rpq, do_block, 0)

    plsc.subcore_barrier()

    obase = cid * n_pad + sid * rpt
    pltpu.sync_copy(agg_sh.at[pl.ds(sid * rpt, rpt)],
                    agg_out.at[pl.ds(obase, rpt)])
    if with_deg:
      pltpu.sync_copy(deg_sh.at[pl.ds(sid * rpt, rpt)],
                      deg_out.at[pl.ds(obase, rpt)])

  return pl.kernel(body, out_type=tuple(out_type), mesh=mesh,
                   scratch_types=scratch)(h, src2, dst2, dummy)


def _tc_embed(c_pad, x_pad, w_embed_pad, n_pad):
  def body(c_ref, x_ref, w_ref, o_ref):
    o_ref[...] = jnp.maximum(
        jnp.dot(c_ref[...], w_ref[...], preferred_element_type=jnp.float32)
        + x_ref[...], 0.0)

  return pl.pallas_call(
      body,
      grid=(n_pad // BN,),
      in_specs=[
          pl.BlockSpec((BN, F), lambda i: (i, 0)),
          pl.BlockSpec((BN, F), lambda i: (i, 0)),
          pl.BlockSpec((F, F), lambda i: (0, 0)),
      ],
      out_specs=pl.BlockSpec((BN, F), lambda i: (i, 0)),
      out_shape=jax.ShapeDtypeStruct((n_pad, F), jnp.float32),
  )(c_pad, x_pad, w_embed_pad)


def _tc_layer(h, p0, p1, d0, d1, w_s, w_n, n_pad):
  def body(h_ref, p0_ref, p1_ref, d0_ref, d1_ref, ws_ref, wn_ref,
           o_ref, inv_ref):
    inv = 1.0 / jnp.maximum(d0_ref[...] + d1_ref[...], 1.0)
    agg = (p0_ref[...] + p1_ref[...]) * inv
    acc = jnp.dot(h_ref[...], ws_ref[...], preferred_element_type=jnp.float32)
    acc = acc + jnp.dot(agg, wn_ref[...], preferred_element_type=jnp.float32)
    o_ref[...] = jnp.maximum(acc, 0.0)
    inv_ref[...] = inv

  row = lambda i: (i, 0)
  full = lambda i: (0, 0)
  return pl.pallas_call(
      body,
      grid=(n_pad // BN,),
      in_specs=[
          pl.BlockSpec((BN, F), row),
          pl.BlockSpec((BN, F), row),
          pl.BlockSpec((BN, F), row),
          pl.BlockSpec((BN, 1), row),
          pl.BlockSpec((BN, 1), row),
          pl.BlockSpec((F, F), full),
          pl.BlockSpec((F, F), full),
      ],
      out_specs=[
          pl.BlockSpec((BN, F), row),
          pl.BlockSpec((BN, 1), row),
      ],
      out_shape=[
          jax.ShapeDtypeStruct((n_pad, F), jnp.float32),
          jax.ShapeDtypeStruct((n_pad, 1), jnp.float32),
      ],
  )(h, p0, p1, d0, d1, w_s, w_n)


def _tc_final(h, p0, p1, inv, w_s, w_n, c_pad, w_out_pad, n_pad):
  def body(h_ref, p0_ref, p1_ref, inv_ref, ws_ref, wn_ref, c_ref, wo_ref,
           o_ref):
    agg = (p0_ref[...] + p1_ref[...]) * inv_ref[...]
    acc = jnp.dot(h_ref[...], ws_ref[...], preferred_element_type=jnp.float32)
    acc = acc + jnp.dot(agg, wn_ref[...], preferred_element_type=jnp.float32)
    h2 = jnp.maximum(acc, 0.0)
    o_ref[...] = c_ref[...] + jnp.dot(
        h2, wo_ref[...], preferred_element_type=jnp.float32)

  row = lambda i: (i, 0)
  full = lambda i: (0, 0)
  return pl.pallas_call(
      body,
      grid=(n_pad // BN,),
      in_specs=[
          pl.BlockSpec((BN, F), row),
          pl.BlockSpec((BN, F), row),
          pl.BlockSpec((BN, F), row),
          pl.BlockSpec((BN, 1), row),
          pl.BlockSpec((F, F), full),
          pl.BlockSpec((F, F), full),
          pl.BlockSpec((BN, F), row),
          pl.BlockSpec((F, F), full),
      ],
      out_specs=pl.BlockSpec((BN, F), lambda i: (i, 0)),
      out_shape=jax.ShapeDtypeStruct((n_pad, F), jnp.float32),
  )(h, p0, p1, inv, w_s, w_n, c_pad, w_out_pad)


def kernel(x, c, edge_index, W_embed, W_self0, W_nbr0, W_self1, W_nbr1, W_out):
  n, f = x.shape
  d = c.shape[1]
  e = edge_index.shape[1]

  n_pad = ((n + BN - 1) // BN) * BN
  nw = 32
  chunk = nw * K * 8 * Q
  e_pad = ((e + chunk - 1) // chunk) * chunk

  f32 = jnp.float32
  x_pad = jnp.zeros((n_pad, F), f32).at[:n].set(x)
  c_pad = jnp.zeros((n_pad, F), f32).at[:n, :d].set(c)
  we_pad = jnp.zeros((F, F), f32).at[:d].set(W_embed)
  wo_pad = jnp.zeros((F, F), f32).at[:, :d].set(W_out)

  pad_e = e_pad - e
  pad_idx = jnp.full((pad_e,), n, jnp.int32)
  src = jnp.concatenate([edge_index[0], pad_idx]).reshape(e_pad // K, K)
  dst = jnp.concatenate([edge_index[1], pad_idx]).reshape(e_pad // K, K)

  dummy = jnp.zeros((K,), f32)
  h0 = _tc_embed(c_pad, x_pad, we_pad, n_pad)
  aggs, degs = _sc_edge_pass(h0, src, dst, dummy, n_pad, with_deg=True)
  d0 = degs[:n_pad, None]
  d1 = degs[n_pad:, None]
  h1, inv = _tc_layer(h0, aggs[:n_pad], aggs[n_pad:], d0, d1,
                      W_self0, W_nbr0, n_pad)
  (aggs1,) = _sc_edge_pass(h1, src, dst, dummy, n_pad, with_deg=False)
  out = _tc_final(h1, aggs1[:n_pad], aggs1[n_pad:], inv,
                  W_self1, W_nbr1, c_pad, wo_pad, n_pad)
  return out[:n, :d]


# speedup vs baseline: 1.4743x; 1.4743x over previous
import functools

import jax
import jax.numpy as jnp
from jax import lax
from jax.experimental import pallas as pl
from jax.experimental.pallas import tpu as pltpu
from jax.experimental.pallas import tpu_sc as plsc

F = 128
BN = 512
K = 128


def _sc_edge_pass(h, src, dst, dummy, n_pad, with_deg):
  info = plsc.get_sparse_core_info()
  nc, ns = info.num_cores, info.num_subcores
  nw = nc * ns
  e_pad = src.shape[0]
  epw = e_pad // nw
  nblk = epw // K
  rpt = n_pad // ns
  mesh = plsc.VectorSubcoreMesh(core_axis_name="c", subcore_axis_name="s")

  out_type = [jax.ShapeDtypeStruct((nc * n_pad, F), jnp.float32)]
  if with_deg:
    out_type.append(jax.ShapeDtypeStruct((nc * n_pad,), jnp.float32))

  scratch = [
      pltpu.VMEM((K,), jnp.int32),
      pltpu.VMEM((K,), jnp.int32),
      pltpu.VMEM((K,), jnp.int32),
      pltpu.VMEM((K,), jnp.int32),
      pltpu.VMEM((K, F), jnp.float32),
      pltpu.VMEM((K, F), jnp.float32),
      pltpu.VMEM((K,), jnp.float32),
      pltpu.VMEM((n_pad // 16,), jnp.float32),
      pltpu.VMEM_SHARED((n_pad, F), jnp.float32),
      pltpu.VMEM_SHARED((n_pad,), jnp.float32),
      pltpu.SemaphoreType.DMA,
      pltpu.SemaphoreType.DMA,
      pltpu.SemaphoreType.DMA,
      pltpu.SemaphoreType.DMA,
  ]

  def body(h_hbm, src_hbm, dst_hbm, dummy_hbm, agg_out, *rest):
    if with_deg:
      deg_out = rest[0]
      rest = rest[1:]
    (idx_s0, idx_d0, idx_s1, idx_d1, rows0, rows1, ones, zdeg, agg_sh,
     deg_sh, gsem0, gsem1, ssem0, ssem1) = rest
    idx_s = (idx_s0, idx_s1)
    idx_d = (idx_d0, idx_d1)
    rows = (rows0, rows1)
    gsem = (gsem0, gsem1)
    ssem = (ssem0, ssem1)
    cid = lax.axis_index("c")
    sid = lax.axis_index("s")
    wid = sid * nc + cid

    zv = jnp.zeros((16,), jnp.float32)

    def z16(i, carry):
      rows0[i // 8, pl.ds((i % 8) * 16, 16)] = zv
      return carry

    lax.fori_loop(0, K * (F // 16), z16, 0)
    for t in range(rpt // K):
      pltpu.sync_copy(rows0, agg_sh.at[pl.ds(sid * rpt + t * K, K)])

    if with_deg:
      ov = jnp.ones((16,), jnp.float32)

      def o16(i, carry):
        ones[pl.ds(i * 16, 16)] = ov
        return carry

      lax.fori_loop(0, K // 16, o16, 0)

      def zd16(i, carry):
        zdeg[pl.ds(i * 16, 16)] = zv
        return carry

      lax.fori_loop(0, rpt // 16, zd16, 0)
      pltpu.sync_copy(zdeg, deg_sh.at[pl.ds(sid * rpt, rpt)])

    plsc.subcore_barrier()

    base = wid * epw

    def idx_load(s, off):
      pltpu.sync_copy(src_hbm.at[pl.ds(off, K)], idx_s[s])
      pltpu.sync_copy(dst_hbm.at[pl.ds(off, K)], idx_d[s])

    def gather_start(s):
      pltpu.async_copy(h_hbm.at[idx_s[s]], rows[s], gsem[s])

    def gather_wait(s):
      pltpu.make_async_copy(h_hbm.at[pl.ds(0, K)], rows[s], gsem[s]).wait()

    def scatter_start(s):
      pltpu.async_copy(rows[s], agg_sh.at[idx_d[s]], ssem[s], add=True)
      if with_deg:
        pltpu.async_copy(ones, deg_sh.at[idx_d[s]], ssem[s], add=True)

    def scatter_wait(s):
      pltpu.make_async_copy(h_hbm.at[pl.ds(0, K)], rows[s], ssem[s]).wait()
      if with_deg:
        pltpu.make_async_copy(dummy_hbm, ones, ssem[s]).wait()

    def step(s, off, wait_other, prefetch):
      o = 1 - s
      gather_wait(s)
      scatter_start(s)
      if prefetch:
        if wait_other:
          scatter_wait(o)
        idx_load(o, off + K)
        gather_start(o)

    idx_load(0, base)
    gather_start(0)
    step(0, base, wait_other=False, prefetch=True)

    def pair(p, carry):
      off = base + (2 * p + 1) * K
      step(1, off, wait_other=True, prefetch=True)
      step(0, off + K, wait_other=True, prefetch=True)
      return carry

    lax.fori_loop(0, (nblk - 2) // 2, pair, 0)
    step(1, base + (nblk - 1) * K, wait_other=False, prefetch=False)
    scatter_wait(0)
    scatter_wait(1)
    plsc.subcore_barrier()

    obase = cid * n_pad + sid * rpt
    pltpu.sync_copy(agg_sh.at[pl.ds(sid * rpt, rpt)],
                    agg_out.at[pl.ds(obase, rpt)])
    if with_deg:
      pltpu.sync_copy(deg_sh.at[pl.ds(sid * rpt, rpt)],
                      deg_out.at[pl.ds(obase, rpt)])

  return pl.kernel(body, out_type=tuple(out_type), mesh=mesh,
                   scratch_types=scratch)(h, src, dst, dummy)


def _tc_embed(c_pad, x_pad, w_embed_pad, n_pad):
  def body(c_ref, x_ref, w_ref, o_ref):
    o_ref[...] = jnp.maximum(
        jnp.dot(c_ref[...], w_ref[...], preferred_element_type=jnp.float32)
        + x_ref[...], 0.0)

  return pl.pallas_call(
      body,
      grid=(n_pad // BN,),
      in_specs=[
          pl.BlockSpec((BN, F), lambda i: (i, 0)),
          pl.BlockSpec((BN, F), lambda i: (i, 0)),
          pl.BlockSpec((F, F), lambda i: (0, 0)),
      ],
      out_specs=pl.BlockSpec((BN, F), lambda i: (i, 0)),
      out_shape=jax.ShapeDtypeStruct((n_pad, F), jnp.float32),
  )(c_pad, x_pad, w_embed_pad)


def _tc_layer(h, p0, p1, d0, d1, w_s, w_n, n_pad):
  def body(h_ref, p0_ref, p1_ref, d0_ref, d1_ref, ws_ref, wn_ref,
           o_ref, inv_ref):
    inv = 1.0 / jnp.maximum(d0_ref[...] + d1_ref[...], 1.0)
    agg = (p0_ref[...] + p1_ref[...]) * inv
    acc = jnp.dot(h_ref[...], ws_ref[...], preferred_element_type=jnp.float32)
    acc = acc + jnp.dot(agg, wn_ref[...], preferred_element_type=jnp.float32)
    o_ref[...] = jnp.maximum(acc, 0.0)
    inv_ref[...] = inv

  row = lambda i: (i, 0)
  full = lambda i: (0, 0)
  return pl.pallas_call(
      body,
      grid=(n_pad // BN,),
      in_specs=[
          pl.BlockSpec((BN, F), row),
          pl.BlockSpec((BN, F), row),
          pl.BlockSpec((BN, F), row),
          pl.BlockSpec((BN, 1), row),
          pl.BlockSpec((BN, 1), row),
          pl.BlockSpec((F, F), full),
          pl.BlockSpec((F, F), full),
      ],
      out_specs=[
          pl.BlockSpec((BN, F), row),
          pl.BlockSpec((BN, 1), row),
      ],
      out_shape=[
          jax.ShapeDtypeStruct((n_pad, F), jnp.float32),
          jax.ShapeDtypeStruct((n_pad, 1), jnp.float32),
      ],
  )(h, p0, p1, d0, d1, w_s, w_n)


def _tc_final(h, p0, p1, inv, w_s, w_n, c_pad, w_out_pad, n_pad):
  def body(h_ref, p0_ref, p1_ref, inv_ref, ws_ref, wn_ref, c_ref, wo_ref,
           o_ref):
    agg = (p0_ref[...] + p1_ref[...]) * inv_ref[...]
    acc = jnp.dot(h_ref[...], ws_ref[...], preferred_element_type=jnp.float32)
    acc = acc + jnp.dot(agg, wn_ref[...], preferred_element_type=jnp.float32)
    h2 = jnp.maximum(acc, 0.0)
    o_ref[...] = c_ref[...] + jnp.dot(
        h2, wo_ref[...], preferred_element_type=jnp.float32)

  row = lambda i: (i, 0)
  full = lambda i: (0, 0)
  return pl.pallas_call(
      body,
      grid=(n_pad // BN,),
      in_specs=[
          pl.BlockSpec((BN, F), row),
          pl.BlockSpec((BN, F), row),
          pl.BlockSpec((BN, F), row),
          pl.BlockSpec((BN, 1), row),
          pl.BlockSpec((F, F), full),
          pl.BlockSpec((F, F), full),
          pl.BlockSpec((BN, F), row),
          pl.BlockSpec((F, F), full),
      ],
      out_specs=pl.BlockSpec((BN, F), lambda i: (i, 0)),
      out_shape=jax.ShapeDtypeStruct((n_pad, F), jnp.float32),
  )(h, p0, p1, inv, w_s, w_n, c_pad, w_out_pad)


def kernel(x, c, edge_index, W_embed, W_self0, W_nbr0, W_self1, W_nbr1, W_out):
  n, f = x.shape
  d = c.shape[1]
  e = edge_index.shape[1]

  n_pad = ((n + BN - 1) // BN) * BN
  nw = 32
  chunk = nw * K
  e_pad = ((e + chunk - 1) // chunk) * chunk

  f32 = jnp.float32
  x_pad = jnp.zeros((n_pad, F), f32).at[:n].set(x)
  c_pad = jnp.zeros((n_pad, F), f32).at[:n, :d].set(c)
  we_pad = jnp.zeros((F, F), f32).at[:d].set(W_embed)
  wo_pad = jnp.zeros((F, F), f32).at[:, :d].set(W_out)

  pad_e = e_pad - e
  pad_idx = jnp.full((pad_e,), n, jnp.int32)
  src = jnp.concatenate([edge_index[0], pad_idx])
  dst = jnp.concatenate([edge_index[1], pad_idx])

  dummy = jnp.zeros((K,), f32)
  h0 = _tc_embed(c_pad, x_pad, we_pad, n_pad)
  aggs, degs = _sc_edge_pass(h0, src, dst, dummy, n_pad, with_deg=True)
  d0 = degs[:n_pad, None]
  d1 = degs[n_pad:, None]
  h1, inv = _tc_layer(h0, aggs[:n_pad], aggs[n_pad:], d0, d1,
                      W_self0, W_nbr0, n_pad)
  (aggs1,) = _sc_edge_pass(h1, src, dst, dummy, n_pad, with_deg=False)
  out = _tc_final(h1, aggs1[:n_pad], aggs1[n_pad:], inv,
                  W_self1, W_nbr1, c_pad, wo_pad, n_pad)
  return out[:n, :d]
